# baseline (speedup 1.0000x reference)
import functools

import jax
import jax.numpy as jnp
from jax.experimental import pallas as pl
from jax.experimental.pallas import tpu as pltpu
from jax.experimental.pallas import tpu_sc as plsc

_KNN_K = 32
_TEMPERATURE = 0.1
_LAMBDA_GEOM = 0.1
_LAMBDA_COMP = 0.1

_CHUNK = 4096
_SUB = 128
_KSEL = 32
_NEG = -3.0e38


def _sc_gather(table, idx, out_rows, width):
    idx2 = idx.reshape(1, out_rows)
    gw = 256
    mesh = plsc.VectorSubcoreMesh(core_axis_name="c", subcore_axis_name="s")

    @pl.kernel(out_type=jax.ShapeDtypeStruct((out_rows, width), table.dtype),
               mesh=mesh)
    def gk(t_hbm, i_hbm, o_hbm):
        def body(i_vmem, o_vmem):
            pltpu.sync_copy(t_hbm.at[i_vmem.at[0]], o_vmem)

        pltpu.emit_pipeline(
            body,
            grid=(out_rows // gw,),
            in_specs=[pl.BlockSpec((1, gw), index_map=lambda i: (0, i))],
            out_specs=[pl.BlockSpec((gw, width), index_map=lambda i: (i, 0))],
            core_axis_name=("c", "s"),
            dimension_semantics=(pltpu.PARALLEL,),
        )(i_hbm, o_hbm)

    return gk(table, idx2)


def kernel(x, mu, alpha, kappa, W_comp, b_comp):
    B, D = x.shape
    N = mu.shape[0]
    nch = -(-N // _CHUNK)
    npad = nch * _CHUNK
    nsub = npad // _SUB
    spc = _CHUNK // _SUB

    def sims_body(x_ref, mu_ref, sims_ref, smax_ref):
        c = pl.program_id(0)
        s = jax.lax.dot_general(
            x_ref[...], mu_ref[...], (((1,), (1,)), ((), ())),
            preferred_element_type=jnp.float32,
            precision=jax.lax.Precision.DEFAULT)
        col = c * _CHUNK + jax.lax.broadcasted_iota(jnp.int32, (B, _CHUNK), 1)
        s = jnp.where(col < N, s, _NEG)
        sims_ref[...] = s
        parts = [jnp.max(s[:, k * _SUB:(k + 1) * _SUB], axis=1, keepdims=True)
                 for k in range(spc)]
        smax_ref[...] = jnp.transpose(jnp.concatenate(parts, axis=1))[None]

    sims, smax = pl.pallas_call(
        sims_body,
        grid=(nch,),
        in_specs=[
            pl.BlockSpec((B, D), lambda c: (0, 0)),
            pl.BlockSpec((_CHUNK, D), lambda c: (c, 0)),
        ],
        out_specs=[
            pl.BlockSpec((B, _CHUNK), lambda c: (0, c)),
            pl.BlockSpec((1, spc, B), lambda c: (c, 0, 0)),
        ],
        out_shape=[
            jax.ShapeDtypeStruct((B, npad), jnp.float32),
            jax.ShapeDtypeStruct((nch, spc, B), jnp.float32),
        ],
    )(x, mu)

    def select_body(smax_ref, flat_ref, sub_ref):
        v = smax_ref[...]
        subid = (jax.lax.broadcasted_iota(jnp.int32, (nch, spc, B), 0) * spc
                 + jax.lax.broadcasted_iota(jnp.int32, (nch, spc, B), 1))
        row = jax.lax.broadcasted_iota(jnp.int32, (_KSEL, B), 1)
        subs = []
        for _ in range(_KSEL):
            m = jnp.max(jnp.max(v, axis=0, keepdims=True), axis=1,
                        keepdims=True)
            sel = jnp.min(jnp.min(jnp.where(v == m, subid, jnp.int32(1 << 30)),
                                  axis=0, keepdims=True),
                          axis=1, keepdims=True)
            subs.append(sel.reshape(1, B))
            v = jnp.where(subid == sel, _NEG, v)
        sub = jnp.concatenate(subs, axis=0)
        sub_ref[...] = sub
        flat_ref[...] = row * nsub + sub

    flat_t, sub_t = pl.pallas_call(
        select_body,
        in_specs=[pl.BlockSpec((nch, spc, B), lambda: (0, 0, 0))],
        out_specs=[pl.BlockSpec((_KSEL, B), lambda: (0, 0)),
                   pl.BlockSpec((_KSEL, B), lambda: (0, 0))],
        out_shape=[jax.ShapeDtypeStruct((_KSEL, B), jnp.int32),
                   jax.ShapeDtypeStruct((_KSEL, B), jnp.int32)],
    )(smax)
    flat = flat_t.T
    sub = sub_t.T

    cand = _sc_gather(sims.reshape(B * nsub, _SUB), flat, B * _KSEL, _SUB)

    ncand = _KSEL * _SUB
    gcol = (sub[:, :, None] * _SUB
            + jnp.arange(_SUB, dtype=jnp.int32)[None, None, :]).reshape(B, ncand)
    Bb = 256
    nrb = B // Bb

    def final_body(cand_ref, gcol_ref, vals_ref, idx_ref):
        v = cand_ref[...]
        g = gcol_ref[...]
        for j in range(_KNN_K):
            m = jnp.max(v, axis=1, keepdims=True)
            eq = v == m
            cg = jnp.min(jnp.where(eq, g, jnp.int32(1 << 30)),
                         axis=1, keepdims=True)
            chosen = eq & (g == cg)
            v = jnp.where(chosen, _NEG, v)
            vals_ref[:, j:j + 1] = m
            idx_ref[:, j:j + 1] = cg

    vals, gidx = pl.pallas_call(
        final_body,
        grid=(nrb,),
        in_specs=[pl.BlockSpec((Bb, ncand), lambda r: (r, 0)),
                  pl.BlockSpec((Bb, ncand), lambda r: (r, 0))],
        out_specs=[pl.BlockSpec((Bb, _KNN_K), lambda r: (r, 0)),
                   pl.BlockSpec((Bb, _KNN_K), lambda r: (r, 0))],
        out_shape=[jax.ShapeDtypeStruct((B, _KNN_K), jnp.float32),
                   jax.ShapeDtypeStruct((B, _KNN_K), jnp.int32)],
    )(cand.reshape(B, ncand), gcol)

    ak = jnp.pad(jnp.stack([alpha, kappa], axis=1), ((0, 0), (0, 126)))
    akg = _sc_gather(ak, gidx, B * _KNN_K, 128)
    alpha_g = akg[:, 0].reshape(B, _KNN_K)
    kappa_g = akg[:, 1].reshape(B, _KNN_K)

    def geom_body(x_ref, eg_ref):
        xv = x_ref[...]
        xx = jax.lax.dot_general(
            xv, xv, (((1,), (1,)), ((), ())),
            preferred_element_type=jnp.float32,
            precision=jax.lax.Precision.DEFAULT)
        ri = jax.lax.broadcasted_iota(jnp.int32, (B, B), 0)
        ci = jax.lax.broadcasted_iota(jnp.int32, (B, B), 1)
        clip = jnp.minimum(xx, 1.0 - 1e-4)
        gv = -jnp.log(1.0 - clip + 1e-4)
        eg_ref[...] = (jnp.sum(jnp.where(ri != ci, gv, 0.0), keepdims=True)
                       / (B * (B - 1)))

    eg = pl.pallas_call(
        geom_body,
        in_specs=[pl.BlockSpec((B, D), lambda: (0, 0))],
        out_specs=pl.BlockSpec((1, 1), lambda: (0, 0)),
        out_shape=jax.ShapeDtypeStruct((1, 1), jnp.float32),
    )(x)

    def combine_body(vals_ref, a_ref, k_ref, eg_ref, W_ref, b_ref, out_ref):
        tv = vals_ref[...]
        al = a_ref[...]
        ka = k_ref[...]
        imp = jnp.maximum(ka, 1e-4)
        w = imp / jnp.sum(imp, axis=1, keepdims=True)
        ex = al * (tv - 1.0) / _TEMPERATURE + jnp.log(jnp.maximum(w, 1e-8))
        m = jnp.max(ex, axis=1, keepdims=True)
        es = -(m + jnp.log(jnp.sum(jnp.exp(ex - m), axis=1, keepdims=True)))

        u = tv[:, 0:1]
        vv = tv[:, 1:2]
        z = (u * W_ref[0, 0] + vv * W_ref[0, 1] + u * vv * W_ref[0, 2]
             + b_ref[0, 0])
        ec = jax.nn.sigmoid(z)

        out_ref[...] = es + _LAMBDA_GEOM * eg_ref[...] + _LAMBDA_COMP * ec

    out = pl.pallas_call(
        combine_body,
        in_specs=[
            pl.BlockSpec((B, _KNN_K), lambda: (0, 0)),
            pl.BlockSpec((B, _KNN_K), lambda: (0, 0)),
            pl.BlockSpec((B, _KNN_K), lambda: (0, 0)),
            pl.BlockSpec((1, 1), lambda: (0, 0)),
            pl.BlockSpec(memory_space=pltpu.SMEM),
            pl.BlockSpec(memory_space=pltpu.SMEM),
        ],
        out_specs=pl.BlockSpec((B, 1), lambda: (0, 0)),
        out_shape=jax.ShapeDtypeStruct((B, 1), jnp.float32),
    )(vals, alpha_g, kappa_g, eg, W_comp, b_comp.reshape(1, 1))

    return out.reshape(B)


# speedup vs baseline: 1.0512x; 1.0512x over previous
import jax
import jax.numpy as jnp
from jax.experimental import pallas as pl
from jax.experimental.pallas import tpu as pltpu
from jax.experimental.pallas import tpu_sc as plsc

_KNN_K = 32
_TEMPERATURE = 0.1
_LAMBDA_GEOM = 0.1
_LAMBDA_COMP = 0.1

_CHUNK = 2048
_SUB = 128
_KSEL = 32
_NEG = -3.0e38


def _sc_gather(table, idx, out_rows, width):
    idx2 = idx.reshape(1, out_rows)
    gw = 256
    mesh = plsc.VectorSubcoreMesh(core_axis_name="c", subcore_axis_name="s")

    @pl.kernel(out_type=jax.ShapeDtypeStruct((out_rows, width), table.dtype),
               mesh=mesh)
    def gk(t_hbm, i_hbm, o_hbm):
        def body(i_vmem, o_vmem):
            pltpu.sync_copy(t_hbm.at[i_vmem.at[0]], o_vmem)

        pltpu.emit_pipeline(
            body,
            grid=(out_rows // gw,),
            in_specs=[pl.BlockSpec((1, gw), index_map=lambda i: (0, i))],
            out_specs=[pl.BlockSpec((gw, width), index_map=lambda i: (i, 0))],
            core_axis_name=("c", "s"),
            dimension_semantics=(pltpu.PARALLEL,),
        )(i_hbm, o_hbm)

    return gk(table, idx2)


def kernel(x, mu, alpha, kappa, W_comp, b_comp):
    B, D = x.shape
    N = mu.shape[0]
    nch = -(-N // _CHUNK)
    npad = nch * _CHUNK
    nsub = npad // _SUB
    spc = _CHUNK // _SUB

    def sims_body(x_ref, mu_ref, sims_ref, smax_ref):
        c = pl.program_id(0)
        s = jax.lax.dot_general(
            x_ref[...], mu_ref[...], (((1,), (1,)), ((), ())),
            preferred_element_type=jnp.float32,
            precision=jax.lax.Precision.DEFAULT)
        col = c * _CHUNK + jax.lax.broadcasted_iota(jnp.int32, (B, _CHUNK), 1)
        s = jnp.where(col < N, s, _NEG)
        sims_ref[...] = s
        parts = [jnp.max(s[:, k * _SUB:(k + 1) * _SUB], axis=1, keepdims=True)
                 for k in range(spc)]
        smax_ref[...] = jnp.transpose(jnp.concatenate(parts, axis=1))[None]

    sims, smax = pl.pallas_call(
        sims_body,
        grid=(nch,),
        in_specs=[
            pl.BlockSpec((B, D), lambda c: (0, 0)),
            pl.BlockSpec((_CHUNK, D), lambda c: (c, 0)),
        ],
        out_specs=[
            pl.BlockSpec((B, _CHUNK), lambda c: (0, c)),
            pl.BlockSpec((1, spc, B), lambda c: (c, 0, 0)),
        ],
        out_shape=[
            jax.ShapeDtypeStruct((B, npad), jnp.float32),
            jax.ShapeDtypeStruct((nch, spc, B), jnp.float32),
        ],
    )(x, mu)

    def select_body(smax_ref, flat_ref, sub_ref):
        v = smax_ref[...]
        subid = (jax.lax.broadcasted_iota(jnp.int32, (nch, spc, B), 0) * spc
                 + jax.lax.broadcasted_iota(jnp.int32, (nch, spc, B), 1))
        row = jax.lax.broadcasted_iota(jnp.int32, (_KSEL, B), 1)
        subs = []
        for _ in range(_KSEL):
            m = jnp.max(jnp.max(v, axis=0, keepdims=True), axis=1,
                        keepdims=True)
            sel = jnp.min(jnp.min(jnp.where(v == m, subid, jnp.int32(1 << 30)),
                                  axis=0, keepdims=True),
                          axis=1, keepdims=True)
            subs.append(sel.reshape(1, B))
            v = jnp.where(subid == sel, _NEG, v)
        sub = jnp.concatenate(subs, axis=0)
        sub_ref[...] = sub
        flat_ref[...] = row * nsub + sub

    flat_t, sub_t = pl.pallas_call(
        select_body,
        in_specs=[pl.BlockSpec((nch, spc, B), lambda: (0, 0, 0))],
        out_specs=[pl.BlockSpec((_KSEL, B), lambda: (0, 0)),
                   pl.BlockSpec((_KSEL, B), lambda: (0, 0))],
        out_shape=[jax.ShapeDtypeStruct((_KSEL, B), jnp.int32),
                   jax.ShapeDtypeStruct((_KSEL, B), jnp.int32)],
    )(smax)
    flat = flat_t.T
    sub = sub_t.T

    cand = _sc_gather(sims.reshape(B * nsub, _SUB), flat, B * _KSEL, _SUB)

    ncand = _KSEL * _SUB
    gcol = (sub[:, :, None] * _SUB
            + jnp.arange(_SUB, dtype=jnp.int32)[None, None, :]).reshape(B, ncand)
    Bb = 256
    nrb = B // Bb

    def final_body(cand_ref, gcol_ref, vals_ref, idx_ref):
        v = cand_ref[...]
        g = gcol_ref[...]
        for j in range(_KNN_K):
            m = jnp.max(v, axis=1, keepdims=True)
            eq = v == m
            cg = jnp.min(jnp.where(eq, g, jnp.int32(1 << 30)),
                         axis=1, keepdims=True)
            chosen = eq & (g == cg)
            v = jnp.where(chosen, _NEG, v)
            vals_ref[:, j:j + 1] = m
            idx_ref[:, j:j + 1] = cg

    vals, gidx = pl.pallas_call(
        final_body,
        grid=(nrb,),
        in_specs=[pl.BlockSpec((Bb, ncand), lambda r: (r, 0)),
                  pl.BlockSpec((Bb, ncand), lambda r: (r, 0))],
        out_specs=[pl.BlockSpec((Bb, _KNN_K), lambda r: (r, 0)),
                   pl.BlockSpec((Bb, _KNN_K), lambda r: (r, 0))],
        out_shape=[jax.ShapeDtypeStruct((B, _KNN_K), jnp.float32),
                   jax.ShapeDtypeStruct((B, _KNN_K), jnp.int32)],
    )(cand.reshape(B, ncand), gcol)

    ak = jnp.pad(jnp.stack([alpha, kappa], axis=1), ((0, 0), (0, 126)))
    akg = _sc_gather(ak, gidx, B * _KNN_K, 128)
    alpha_g = akg[:, 0].reshape(B, _KNN_K)
    kappa_g = akg[:, 1].reshape(B, _KNN_K)

    def geom_body(x_ref, eg_ref):
        xv = x_ref[...]
        xx = jax.lax.dot_general(
            xv, xv, (((1,), (1,)), ((), ())),
            preferred_element_type=jnp.float32,
            precision=jax.lax.Precision.DEFAULT)
        ri = jax.lax.broadcasted_iota(jnp.int32, (B, B), 0)
        ci = jax.lax.broadcasted_iota(jnp.int32, (B, B), 1)
        clip = jnp.minimum(xx, 1.0 - 1e-4)
        gv = -jnp.log(1.0 - clip + 1e-4)
        eg_ref[...] = (jnp.sum(jnp.where(ri != ci, gv, 0.0), keepdims=True)
                       / (B * (B - 1)))

    eg = pl.pallas_call(
        geom_body,
        in_specs=[pl.BlockSpec((B, D), lambda: (0, 0))],
        out_specs=pl.BlockSpec((1, 1), lambda: (0, 0)),
        out_shape=jax.ShapeDtypeStruct((1, 1), jnp.float32),
    )(x)

    def combine_body(vals_ref, a_ref, k_ref, eg_ref, W_ref, b_ref, out_ref):
        tv = vals_ref[...]
        al = a_ref[...]
        ka = k_ref[...]
        imp = jnp.maximum(ka, 1e-4)
        w = imp / jnp.sum(imp, axis=1, keepdims=True)
        ex = al * (tv - 1.0) / _TEMPERATURE + jnp.log(jnp.maximum(w, 1e-8))
        m = jnp.max(ex, axis=1, keepdims=True)
        es = -(m + jnp.log(jnp.sum(jnp.exp(ex - m), axis=1, keepdims=True)))

        u = tv[:, 0:1]
        vv = tv[:, 1:2]
        z = (u * W_ref[0, 0] + vv * W_ref[0, 1] + u * vv * W_ref[0, 2]
             + b_ref[0, 0])
        ec = jax.nn.sigmoid(z)

        out_ref[...] = es + _LAMBDA_GEOM * eg_ref[...] + _LAMBDA_COMP * ec

    out = pl.pallas_call(
        combine_body,
        in_specs=[
            pl.BlockSpec((B, _KNN_K), lambda: (0, 0)),
            pl.BlockSpec((B, _KNN_K), lambda: (0, 0)),
            pl.BlockSpec((B, _KNN_K), lambda: (0, 0)),
            pl.BlockSpec((1, 1), lambda: (0, 0)),
            pl.BlockSpec(memory_space=pltpu.SMEM),
            pl.BlockSpec(memory_space=pltpu.SMEM),
        ],
        out_specs=pl.BlockSpec((B, 1), lambda: (0, 0)),
        out_shape=jax.ShapeDtypeStruct((B, 1), jnp.float32),
    )(vals, alpha_g, kappa_g, eg, W_comp, b_comp.reshape(1, 1))

    return out.reshape(B)
